# baseline (speedup 1.0000x reference)
import functools

import jax
import jax.numpy as jnp
from jax import lax
from jax.experimental import pallas as pl
from jax.experimental.pallas import tpu as pltpu
from jax.experimental.pallas import tpu_sc as plsc

N = 10000
D = 128
E = 320000
EPS = 1e-5

NC = 2
NS = 16
NW = NC * NS
B = 128
NBLK = -(-E // (NW * B))
EPW = NBLK * B
EPAD = EPW * NW
NPAD = 10240
RPT = NPAD // NS
GARBAGE = NPAD - N

_mesh = plsc.VectorSubcoreMesh(
    core_axis_name="c", subcore_axis_name="s", num_cores=NC, num_subcores=NS
)


@functools.partial(
    pl.kernel,
    out_type=jax.ShapeDtypeStruct((NC, NPAD, 16), jnp.float32),
    mesh=_mesh,
    scratch_types=[
        pltpu.VMEM((NBLK, B), jnp.int32),
        pltpu.VMEM((B, 16), jnp.float32),
        pltpu.VMEM((B, 16), jnp.float32),
        pltpu.VMEM_SHARED((NPAD, 16), jnp.float32),
    ],
)
def _deg_kernel(dst_hbm, degp_hbm, idx_v, ones_v, zb_v, deg_sh):
    c = lax.axis_index("c")
    s = lax.axis_index("s")
    wid = c * NS + s

    @pl.loop(0, B)
    def _fill(i):
        ones_v[i] = jnp.ones((16,), jnp.float32)
        zb_v[i] = jnp.zeros((16,), jnp.float32)

    base = s * RPT

    @pl.loop(0, RPT // B)
    def _zero(j):
        pltpu.sync_copy(zb_v, deg_sh.at[pl.ds(base + j * B, B)])

    plsc.subcore_barrier()
    pltpu.sync_copy(dst_hbm.at[wid], idx_v)

    @pl.loop(0, NBLK)
    def _acc(j):
        pltpu.sync_copy(ones_v, deg_sh.at[idx_v.at[j]], add=True)

    plsc.subcore_barrier()
    pltpu.sync_copy(deg_sh.at[pl.ds(base, RPT)], degp_hbm.at[c, pl.ds(base, RPT)])


@functools.partial(
    pl.kernel,
    out_type=jax.ShapeDtypeStruct((NC, NPAD, D), jnp.float32),
    mesh=_mesh,
    scratch_types=[
        pltpu.VMEM((NBLK, B), jnp.int32),
        pltpu.VMEM((NBLK, B), jnp.int32),
        pltpu.VMEM((B, D), jnp.float32),
        pltpu.VMEM_SHARED((NPAD, D), jnp.float32),
    ],
)
def _edge_kernel(hp_hbm, src_hbm, dst_hbm, accp_hbm, isrc_v, idst_v, rows_v, acc_sh):
    c = lax.axis_index("c")
    s = lax.axis_index("s")
    wid = c * NS + s

    @pl.loop(0, B)
    def _fill(i):
        for k in range(D // 16):
            rows_v[i, pl.ds(k * 16, 16)] = jnp.zeros((16,), jnp.float32)

    base = s * RPT

    @pl.loop(0, RPT // B)
    def _zero(j):
        pltpu.sync_copy(rows_v, acc_sh.at[pl.ds(base + j * B, B)])

    plsc.subcore_barrier()
    pltpu.sync_copy(src_hbm.at[wid], isrc_v)
    pltpu.sync_copy(dst_hbm.at[wid], idst_v)

    @pl.loop(0, NBLK)
    def _acc(j):
        pltpu.sync_copy(hp_hbm.at[isrc_v.at[j]], rows_v)
        pltpu.sync_copy(rows_v, acc_sh.at[idst_v.at[j]], add=True)

    plsc.subcore_barrier()
    pltpu.sync_copy(acc_sh.at[pl.ds(base, RPT)], accp_hbm.at[c, pl.ds(base, RPT)])


R = 2000
GRID = N // R


def _tc_first_body(degp_ref, x_ref, w_ref, dis_ref, hp_ref):
    deg = degp_ref[0, :, 0:1] + degp_ref[1, :, 0:1] + 1.0
    dis = lax.rsqrt(deg)
    dis_ref[...] = dis
    h = jnp.dot(x_ref[...], w_ref[...], preferred_element_type=jnp.float32)
    hp_ref[...] = h * dis


_tc_first = pl.pallas_call(
    _tc_first_body,
    grid=(GRID,),
    in_specs=[
        pl.BlockSpec((NC, R, 16), lambda i: (0, i, 0)),
        pl.BlockSpec((R, D), lambda i: (i, 0)),
        pl.BlockSpec((D, D), lambda i: (0, 0)),
    ],
    out_specs=[
        pl.BlockSpec((R, 1), lambda i: (i, 0)),
        pl.BlockSpec((R, D), lambda i: (i, 0)),
    ],
    out_shape=[
        jax.ShapeDtypeStruct((N, 1), jnp.float32),
        jax.ShapeDtypeStruct((N, D), jnp.float32),
    ],
)


def _tc_layer_body(accp_ref, hp_ref, dis_ref, b_ref, g_ref, be_ref, m_ref, v_ref,
                   w_ref, out_ref):
    dis = dis_ref[...]
    t = (accp_ref[0] + accp_ref[1] + hp_ref[...]) * dis + b_ref[...]
    t = (t - m_ref[...]) * lax.rsqrt(v_ref[...] + EPS) * g_ref[...] + be_ref[...]
    t = jnp.maximum(t, 0.0)
    out_ref[...] = jnp.dot(t, w_ref[...], preferred_element_type=jnp.float32) * dis


_vec = pl.BlockSpec((1, D), lambda i: (0, 0))
_mat = pl.BlockSpec((D, D), lambda i: (0, 0))
_rowblk = pl.BlockSpec((R, D), lambda i: (i, 0))
_accblk = pl.BlockSpec((NC, R, D), lambda i: (0, i, 0))
_disblk = pl.BlockSpec((R, 1), lambda i: (i, 0))

_tc_layer = pl.pallas_call(
    _tc_layer_body,
    grid=(GRID,),
    in_specs=[_accblk, _rowblk, _disblk, _vec, _vec, _vec, _vec, _vec, _mat],
    out_specs=_rowblk,
    out_shape=jax.ShapeDtypeStruct((N, D), jnp.float32),
)


def _tc_final_body(accp_ref, hp_ref, dis_ref, b_ref, wm1_ref, bm1_ref,
                   wm2_ref, bm2_ref, out_ref):
    t = (accp_ref[0] + accp_ref[1] + hp_ref[...]) * dis_ref[...] + b_ref[...]
    z = jnp.dot(t, wm1_ref[...], preferred_element_type=jnp.float32) + bm1_ref[...]
    z = jnp.maximum(z, 0.0)
    out_ref[...] = jnp.dot(z, wm2_ref[...], preferred_element_type=jnp.float32) + bm2_ref[...]


_tc_final = pl.pallas_call(
    _tc_final_body,
    grid=(GRID,),
    in_specs=[_accblk, _rowblk, _disblk, _vec, _mat, _vec, _mat, _vec],
    out_specs=_rowblk,
    out_shape=jax.ShapeDtypeStruct((N, D), jnp.float32),
)


def kernel(x, edge_index, W0, b0, g0, be0, m0, v0, W1, b1, g1, be1, m1, v1,
           W2, b2, Wm1, bm1, Wm2, bm2):
    src = edge_index[0].astype(jnp.int32)
    dst = edge_index[1].astype(jnp.int32)
    npad = EPAD - E
    ar = jnp.arange(npad, dtype=jnp.int32)
    src_p = jnp.concatenate([src, (ar * 131) % N]).reshape(NW, NBLK, B)
    dst_p = jnp.concatenate([dst, N + ar % GARBAGE]).reshape(NW, NBLK, B)

    r1 = lambda a: a.reshape(1, D)
    degp = _deg_kernel(dst_p)
    dis, h0p = _tc_first(degp, x, W0)
    acc0 = _edge_kernel(h0p, src_p, dst_p)
    h1p = _tc_layer(acc0, h0p, dis, r1(b0), r1(g0), r1(be0), r1(m0), r1(v0), W1)
    acc1 = _edge_kernel(h1p, src_p, dst_p)
    h2p = _tc_layer(acc1, h1p, dis, r1(b1), r1(g1), r1(be1), r1(m1), r1(v1), W2)
    acc2 = _edge_kernel(h2p, src_p, dst_p)
    return _tc_final(acc2, h2p, dis, r1(b2), Wm1, r1(bm1), Wm2, r1(bm2))
